# baseline (speedup 1.0000x reference)
import functools

import jax
import jax.numpy as jnp
from jax import lax
from jax.experimental import pallas as pl
from jax.experimental.pallas import tpu as pltpu
from jax.experimental.pallas import tpu_sc as plsc

_NC = 2
_NS = 16
_NW = _NC * _NS


def _gather_fn(N, D, C, NBUF, TAIL):
    b_per_w = N // _NW
    sizes = [C] * (b_per_w // C - 1) + [TAIL] * (C // TAIL)
    offs = [sum(sizes[:i]) for i in range(len(sizes))]
    n_chunks = len(sizes)
    mesh = plsc.VectorSubcoreMesh(core_axis_name="c", subcore_axis_name="s")

    @functools.partial(
        pl.kernel,
        mesh=mesh,
        out_type=jax.ShapeDtypeStruct((N, D), jnp.float32),
        scratch_types=[
            pltpu.VMEM((b_per_w,), jnp.int32),
            pltpu.VMEM((NBUF, C, D), jnp.float32),
        ]
        + [pltpu.SemaphoreType.DMA] * (2 * NBUF),
        compiler_params=pltpu.CompilerParams(use_tc_tiling_on_sc=False),
    )
    def k(table_hbm, idx_hbm, out_hbm, idx_v, rows_v, *sems):
        g_sems = sems[:NBUF]
        s_sems = sems[NBUF:]
        wid = lax.axis_index("s") * _NC + lax.axis_index("c")
        base = wid * b_per_w
        pltpu.sync_copy(idx_hbm.at[pl.ds(base, b_per_w)], idx_v)

        gathers = [None] * NBUF
        stores = [None] * NBUF
        for j in range(min(NBUF - 1, n_chunks)):
            gathers[j] = pltpu.async_copy(
                table_hbm.at[idx_v.at[pl.ds(offs[j], sizes[j])]],
                rows_v.at[j, pl.ds(0, sizes[j])],
                g_sems[j],
            )
        for i in range(n_chunks):
            b = i % NBUF
            pre = i + NBUF - 1
            if pre < n_chunks:
                pb = pre % NBUF
                if stores[pb] is not None:
                    stores[pb].wait()
                gathers[pb] = pltpu.async_copy(
                    table_hbm.at[idx_v.at[pl.ds(offs[pre], sizes[pre])]],
                    rows_v.at[pb, pl.ds(0, sizes[pre])],
                    g_sems[pb],
                )
            gathers[b].wait()
            stores[b] = pltpu.async_copy(
                rows_v.at[b, pl.ds(0, sizes[i])],
                out_hbm.at[pl.ds(base + offs[i], sizes[i])],
                s_sems[b],
            )
        for st in stores:
            if st is not None:
                st.wait()

    return k


def kernel(weight, words):
    B, H = words.shape
    V, D = weight.shape
    N = B * H
    flat = words.reshape(N).astype(jnp.int32)
    C = 1600
    NBUF = 2
    TAIL = 200
    out = _gather_fn(N, D, C, NBUF, TAIL)(weight, flat)
    return out.reshape(B, H, D)


# speedup vs baseline: 1.0008x; 1.0008x over previous
import functools

import jax
import jax.numpy as jnp
from jax import lax
from jax.experimental import pallas as pl
from jax.experimental.pallas import tpu as pltpu
from jax.experimental.pallas import tpu_sc as plsc

_NC = 2
_NS = 16
_NW = _NC * _NS


def _gather_fn(N, D, C, NBUF, TAIL):
    b_per_w = N // _NW
    sizes = [C] * (b_per_w // C - 1) + [TAIL] * (C // TAIL)
    offs = [sum(sizes[:i]) for i in range(len(sizes))]
    n_chunks = len(sizes)
    mesh = plsc.VectorSubcoreMesh(core_axis_name="c", subcore_axis_name="s")

    @functools.partial(
        pl.kernel,
        mesh=mesh,
        out_type=jax.ShapeDtypeStruct((N, D), jnp.float32),
        scratch_types=[
            pltpu.VMEM((b_per_w,), jnp.int32),
            pltpu.VMEM((NBUF, C, D), jnp.float32),
        ]
        + [pltpu.SemaphoreType.DMA] * (2 * NBUF),
        compiler_params=pltpu.CompilerParams(use_tc_tiling_on_sc=False),
    )
    def k(table_hbm, idx_hbm, out_hbm, idx_v, rows_v, *sems):
        g_sems = sems[:NBUF]
        s_sems = sems[NBUF:]
        wid = lax.axis_index("s") * _NC + lax.axis_index("c")
        base = wid * b_per_w
        pltpu.sync_copy(idx_hbm.at[pl.ds(base, b_per_w)], idx_v)

        gathers = [None] * NBUF
        stores = [None] * NBUF
        for j in range(min(NBUF - 1, n_chunks)):
            gathers[j] = pltpu.async_copy(
                table_hbm.at[idx_v.at[pl.ds(offs[j], sizes[j])]],
                rows_v.at[j, pl.ds(0, sizes[j])],
                g_sems[j],
            )
        for i in range(n_chunks):
            b = i % NBUF
            pre = i + NBUF - 1
            if pre < n_chunks:
                pb = pre % NBUF
                if stores[pb] is not None:
                    stores[pb].wait()
                gathers[pb] = pltpu.async_copy(
                    table_hbm.at[idx_v.at[pl.ds(offs[pre], sizes[pre])]],
                    rows_v.at[pb, pl.ds(0, sizes[pre])],
                    g_sems[pb],
                )
            gathers[b].wait()
            stores[b] = pltpu.async_copy(
                rows_v.at[b, pl.ds(0, sizes[i])],
                out_hbm.at[pl.ds(base + offs[i], sizes[i])],
                s_sems[b],
            )
        for st in stores:
            if st is not None:
                st.wait()

    return k


def kernel(weight, words):
    B, H = words.shape
    V, D = weight.shape
    N = B * H
    flat = words.reshape(N).astype(jnp.int32)
    C = 1600
    NBUF = 2
    TAIL = 1600
    out = _gather_fn(N, D, C, NBUF, TAIL)(weight, flat)
    return out.reshape(B, H, D)
